# baseline (speedup 1.0000x reference)
import functools
import math

import jax
import jax.numpy as jnp
from jax import lax
from jax.experimental import pallas as pl
from jax.experimental.pallas import tpu as pltpu
from jax.experimental.pallas import tpu_sc as plsc

CH = 128
LANES = 16


def _sc_mesh_info():
  info = plsc.get_sparse_core_info()
  return info.num_cores, info.num_subcores


def _sc_degree(dst_idx, npad):
  nc, ns = _sc_mesh_info()
  nw, nch, _ = dst_idx.shape
  rows_per_tile = npad // ns
  GRP = 12
  mesh = plsc.VectorSubcoreMesh(core_axis_name="c", subcore_axis_name="s")

  @functools.partial(
      pl.kernel,
      out_type=jax.ShapeDtypeStruct((nc, npad), jnp.float32),
      mesh=mesh,
      scratch_types=[
          pltpu.VMEM((nch, CH), jnp.int32),
          pltpu.VMEM((CH,), jnp.float32),
          pltpu.VMEM((rows_per_tile,), jnp.float32),
          pltpu.VMEM_SHARED((npad,), jnp.float32),
          pltpu.SemaphoreType.DMA,
      ],
  )
  def k(didx_hbm, out_hbm, didx_v, vals_v, z_v, acc_sh, sem):
    c = lax.axis_index("c")
    s = lax.axis_index("s")
    wid = s * nc + c
    pltpu.sync_copy(didx_hbm.at[wid], didx_v)
    for i in range(0, CH, LANES):
      vals_v[pl.ds(i, LANES)] = jnp.ones((LANES,), jnp.float32)
    for i in range(0, rows_per_tile, LANES):
      z_v[pl.ds(i, LANES)] = jnp.zeros((LANES,), jnp.float32)
    row0 = s * rows_per_tile
    pltpu.sync_copy(z_v, acc_sh.at[pl.ds(row0, rows_per_tile)])
    plsc.subcore_barrier()

    def body(g, carry):
      j0 = GRP * g
      for q in range(GRP):
        pltpu.async_copy(vals_v, acc_sh.at[didx_v.at[j0 + q]], sem, add=True)
      for q in range(GRP):
        pltpu.make_async_copy(vals_v, acc_sh.at[didx_v.at[0]], sem).wait()
      return carry

    lax.fori_loop(0, nch // GRP, body, 0)
    for j in range(nch - nch % GRP, nch):
      pltpu.sync_copy(vals_v, acc_sh.at[didx_v.at[j]], add=True)
    plsc.subcore_barrier()
    pltpu.sync_copy(acc_sh.at[pl.ds(row0, rows_per_tile)],
                    out_hbm.at[c, pl.ds(row0, rows_per_tile)])

  return k(dst_idx)


def _sc_segsum_1d(vals, src_idx, dst_idx, npad):
  nc, ns = _sc_mesh_info()
  nw, nch_a, _ = dst_idx.shape
  rows_per_tile = npad // ns
  mesh = plsc.VectorSubcoreMesh(core_axis_name="c", subcore_axis_name="s")

  @functools.partial(
      pl.kernel,
      out_type=jax.ShapeDtypeStruct((nc, npad), jnp.float32),
      mesh=mesh,
      scratch_types=[
          pltpu.VMEM((nch_a, CH), jnp.int32),
          pltpu.VMEM((nch_a, CH), jnp.int32),
          pltpu.VMEM((4, CH), jnp.float32),
          pltpu.VMEM((rows_per_tile,), jnp.float32),
          pltpu.VMEM_SHARED((npad,), jnp.float32),
          pltpu.SemaphoreType.DMA,
          pltpu.SemaphoreType.DMA,
          pltpu.SemaphoreType.DMA,
          pltpu.SemaphoreType.DMA,
      ],
  )
  def k(vals_hbm, sidx_hbm, didx_hbm, out_hbm, sidx_v, didx_v, bufs, z_v,
        acc_sh, *gsems):
    c = lax.axis_index("c")
    s = lax.axis_index("s")
    wid = s * nc + c
    pltpu.sync_copy(sidx_hbm.at[wid], sidx_v)
    pltpu.sync_copy(didx_hbm.at[wid], didx_v)
    for i in range(0, rows_per_tile, LANES):
      z_v[pl.ds(i, LANES)] = jnp.zeros((LANES,), jnp.float32)
    row0 = s * rows_per_tile
    pltpu.sync_copy(z_v, acc_sh.at[pl.ds(row0, rows_per_tile)])
    plsc.subcore_barrier()

    for q in range(4):
      pltpu.async_copy(vals_hbm.at[sidx_v.at[q]], bufs.at[q], gsems[q])

    def body(k4, carry):
      j0 = 4 * k4
      for q in range(4):
        pltpu.make_async_copy(
            vals_hbm.at[sidx_v.at[0]], bufs.at[q], gsems[q]).wait()
        pltpu.sync_copy(bufs.at[q], acc_sh.at[didx_v.at[j0 + q]], add=True)
        nxt = jnp.minimum(j0 + 4 + q, nch_a - 1)
        pltpu.async_copy(vals_hbm.at[sidx_v.at[nxt]], bufs.at[q], gsems[q])
      return carry

    lax.fori_loop(0, nch_a // 4, body, 0)
    for q in range(4):
      pltpu.make_async_copy(
          vals_hbm.at[sidx_v.at[0]], bufs.at[q], gsems[q]).wait()
    plsc.subcore_barrier()
    pltpu.sync_copy(acc_sh.at[pl.ds(row0, rows_per_tile)],
                    out_hbm.at[c, pl.ds(row0, rows_per_tile)])

  return k(vals, src_idx, dst_idx)


def _sc_segsum_rows(table, idx, npad, d, nch):
  nc, ns = _sc_mesh_info()
  nw, nch_a, two, ch = idx.shape
  rows_per_tile = npad // ns
  zrows = 16
  mesh = plsc.VectorSubcoreMesh(core_axis_name="c", subcore_axis_name="s")

  @functools.partial(
      pl.kernel,
      out_type=jax.ShapeDtypeStruct((nc, npad, d), jnp.float32),
      mesh=mesh,
      scratch_types=[
          pltpu.VMEM((4, 2, ch), jnp.int32),
          pltpu.VMEM((ch, d), jnp.float32),
          pltpu.VMEM((ch, d), jnp.float32),
          pltpu.VMEM((zrows, d), jnp.float32),
          pltpu.VMEM_SHARED((npad, d), jnp.float32),
          pltpu.SemaphoreType.DMA,
          pltpu.SemaphoreType.DMA,
          pltpu.SemaphoreType.DMA,
      ],
  )
  def k(tab_hbm, idx_hbm, out_hbm, islot, buf0, buf1, z_v, acc_sh,
        gsem0, gsem1, isem):
    c = lax.axis_index("c")
    s = lax.axis_index("s")
    wid = s * nc + c
    for r in range(zrows):
      for i in range(0, d, LANES):
        z_v[r, pl.ds(i, LANES)] = jnp.zeros((LANES,), jnp.float32)
    row0 = s * rows_per_tile

    def zbody(t, carry):
      pltpu.sync_copy(z_v, acc_sh.at[pl.ds(row0 + t * zrows, zrows)])
      return carry

    lax.fori_loop(0, rows_per_tile // zrows, zbody, 0)
    plsc.subcore_barrier()

    for q in range(4):
      pltpu.sync_copy(idx_hbm.at[wid, q], islot.at[q])
    pltpu.async_copy(tab_hbm.at[islot.at[0, 0]], buf0, gsem0)
    pltpu.async_copy(tab_hbm.at[islot.at[1, 0]], buf1, gsem1)

    def body(k2, carry):
      j0 = 2 * k2
      s0 = lax.rem(j0, 4)
      s1 = lax.rem(j0 + 1, 4)
      s2 = lax.rem(j0 + 2, 4)
      s3 = lax.rem(j0 + 3, 4)
      pltpu.make_async_copy(tab_hbm.at[islot.at[s0, 0]], buf0, gsem0).wait()
      pltpu.sync_copy(buf0, acc_sh.at[islot.at[s0, 1]], add=True)
      pltpu.async_copy(idx_hbm.at[wid, j0 + 4], islot.at[s0], isem)
      pltpu.async_copy(tab_hbm.at[islot.at[s2, 0]], buf0, gsem0)
      pltpu.make_async_copy(tab_hbm.at[islot.at[s1, 0]], buf1, gsem1).wait()
      pltpu.sync_copy(buf1, acc_sh.at[islot.at[s1, 1]], add=True)
      pltpu.async_copy(idx_hbm.at[wid, j0 + 5], islot.at[s1], isem)
      pltpu.async_copy(tab_hbm.at[islot.at[s3, 0]], buf1, gsem1)
      pltpu.make_async_copy(idx_hbm.at[wid, 0], islot.at[s0], isem).wait()
      pltpu.make_async_copy(idx_hbm.at[wid, 0], islot.at[s1], isem).wait()
      return carry

    lax.fori_loop(0, nch // 2, body, 0)
    pltpu.make_async_copy(tab_hbm.at[islot.at[0, 0]], buf0, gsem0).wait()
    pltpu.make_async_copy(tab_hbm.at[islot.at[1, 0]], buf1, gsem1).wait()
    plsc.subcore_barrier()
    pltpu.sync_copy(acc_sh.at[pl.ds(row0, rows_per_tile)],
                    out_hbm.at[c, pl.ds(row0, rows_per_tile)])

  return k(table, idx)


def _sc_segsum_rows_serial(table, src_idx, dst_idx, npad, d):
  nc, ns = _sc_mesh_info()
  nw, nch, _ = dst_idx.shape
  rows_per_tile = npad // ns
  zrows = 16
  mesh = plsc.VectorSubcoreMesh(core_axis_name="c", subcore_axis_name="s")

  @functools.partial(
      pl.kernel,
      out_type=jax.ShapeDtypeStruct((nc, npad, d), jnp.float32),
      mesh=mesh,
      scratch_types=[
          pltpu.VMEM((nch, CH), jnp.int32),
          pltpu.VMEM((nch, CH), jnp.int32),
          pltpu.VMEM((CH, d), jnp.float32),
          pltpu.VMEM((zrows, d), jnp.float32),
          pltpu.VMEM_SHARED((npad, d), jnp.float32),
          pltpu.SemaphoreType.DMA,
      ],
  )
  def k(tab_hbm, sidx_hbm, didx_hbm, out_hbm, sidx_v, didx_v, rows_v, z_v,
        acc_sh, sem):
    c = lax.axis_index("c")
    s = lax.axis_index("s")
    wid = s * nc + c
    pltpu.sync_copy(sidx_hbm.at[wid], sidx_v)
    pltpu.sync_copy(didx_hbm.at[wid], didx_v)
    for r in range(zrows):
      for i in range(0, d, LANES):
        z_v[r, pl.ds(i, LANES)] = jnp.zeros((LANES,), jnp.float32)
    row0 = s * rows_per_tile

    def zbody(t, carry):
      pltpu.sync_copy(z_v, acc_sh.at[pl.ds(row0 + t * zrows, zrows)])
      return carry

    lax.fori_loop(0, rows_per_tile // zrows, zbody, 0)
    plsc.subcore_barrier()

    def body(j, carry):
      pltpu.async_copy(tab_hbm.at[sidx_v.at[j]], rows_v, sem).wait()
      pltpu.sync_copy(rows_v, acc_sh.at[didx_v.at[j]], add=True)
      return carry

    lax.fori_loop(0, nch, body, 0)
    plsc.subcore_barrier()
    pltpu.sync_copy(acc_sh.at[pl.ds(row0, rows_per_tile)],
                    out_hbm.at[c, pl.ds(row0, rows_per_tile)])

  return k(table, src_idx, dst_idx)


def _tc_matmul(x_pad, w1, blk):
  npad, d_in = x_pad.shape
  d_hid = w1.shape[1]
  grid = npad // blk

  def body(xb, wb, mb):
    mb[...] = jnp.dot(xb[...], wb[...], preferred_element_type=jnp.float32)

  return pl.pallas_call(
      body,
      grid=(grid,),
      in_specs=[
          pl.BlockSpec((blk, d_in), lambda i: (i, 0)),
          pl.BlockSpec((d_in, d_hid), lambda i: (0, 0)),
      ],
      out_specs=pl.BlockSpec((blk, d_hid), lambda i: (i, 0)),
      out_shape=jax.ShapeDtypeStruct((npad, d_hid), jnp.float32),
  )(x_pad, w1)


def _tc_scale(m1, degp, blk):
  npad, d_hid = m1.shape
  nc = degp.shape[0]
  grid = npad // blk

  def body(mb, degb, g1b, dinvb):
    deg = degb[0] + degb[1] + 1.0
    dinv = lax.rsqrt(deg)
    g1b[...] = mb[...] * dinv
    dinvb[...] = dinv

  return pl.pallas_call(
      body,
      grid=(grid,),
      in_specs=[
          pl.BlockSpec((blk, d_hid), lambda i: (i, 0)),
          pl.BlockSpec((nc, blk, 1), lambda i: (0, i, 0)),
      ],
      out_specs=[
          pl.BlockSpec((blk, d_hid), lambda i: (i, 0)),
          pl.BlockSpec((blk, 1), lambda i: (i, 0)),
      ],
      out_shape=[
          jax.ShapeDtypeStruct((npad, d_hid), jnp.float32),
          jax.ShapeDtypeStruct((npad, 1), jnp.float32),
      ],
  )(m1, degp)


def _tc_layer2_in(p1, g1, dinv, b1, w2, blk):
  nc, npad, d_hid = p1.shape
  d_out = w2.shape[1]
  grid = npad // blk

  def body(pb, g1b, dinvb, b1b, wb, g2b):
    s = (pb[0] + pb[1] + g1b[...]) * dinvb[...]
    h = jnp.maximum(s + b1b[...], 0.0)
    mm = jnp.dot(h, wb[...], preferred_element_type=jnp.float32)
    g2b[...] = mm * dinvb[...]

  return pl.pallas_call(
      body,
      grid=(grid,),
      in_specs=[
          pl.BlockSpec((nc, blk, d_hid), lambda i: (0, i, 0)),
          pl.BlockSpec((blk, d_hid), lambda i: (i, 0)),
          pl.BlockSpec((blk, 1), lambda i: (i, 0)),
          pl.BlockSpec((1, d_hid), lambda i: (0, 0)),
          pl.BlockSpec((d_hid, d_out), lambda i: (0, 0)),
      ],
      out_specs=pl.BlockSpec((blk, d_out), lambda i: (i, 0)),
      out_shape=jax.ShapeDtypeStruct((npad, d_out), jnp.float32),
  )(p1, g1, dinv, b1, w2)


def _tc_combine(p2, g2, dinv, b2, blk):
  nc, npad, d_out = p2.shape
  grid = npad // blk

  def body(pb, g2b, dinvb, b2b, outb):
    outb[...] = (pb[0] + pb[1] + g2b[...]) * dinvb[...] + b2b[...]

  return pl.pallas_call(
      body,
      grid=(grid,),
      in_specs=[
          pl.BlockSpec((nc, blk, d_out), lambda i: (0, i, 0)),
          pl.BlockSpec((blk, d_out), lambda i: (i, 0)),
          pl.BlockSpec((blk, 1), lambda i: (i, 0)),
          pl.BlockSpec((1, 1), lambda i: (0, 0)),
      ],
      out_specs=pl.BlockSpec((blk, d_out), lambda i: (i, 0)),
      out_shape=jax.ShapeDtypeStruct((npad, d_out), jnp.float32),
  )(p2, g2, dinv, b2)


def kernel(x, edge_index, W1, b1, W2, b2):
  n, d_in = x.shape
  d_hid = W1.shape[1]
  d_out = W2.shape[1]
  e = edge_index.shape[1]
  nc, ns = _sc_mesh_info()
  nw = nc * ns

  blk = 1024
  npad = ((n + 1 + blk - 1) // blk) * blk
  dump = n

  nch = math.ceil(e / (nw * CH))
  nch = ((nch + 3) // 4) * 4
  e_pad = nw * nch * CH
  src = edge_index[0].astype(jnp.int32)
  dst = edge_index[1].astype(jnp.int32)
  pad_n = e_pad - e
  pad_src = jnp.arange(pad_n, dtype=jnp.int32) % n
  pad_dst = n + jnp.arange(pad_n, dtype=jnp.int32) % (npad - n)
  la_n = nw * 4 * CH
  la_src = (jnp.arange(la_n, dtype=jnp.int32) % n).reshape(nw, 4, CH)
  la_dst = (n + jnp.arange(la_n, dtype=jnp.int32) % (npad - n)).reshape(
      nw, 4, CH)
  src_p = jnp.concatenate([src, pad_src]).reshape(nw, nch, CH)
  dst_p = jnp.concatenate([dst, pad_dst]).reshape(nw, nch, CH)
  src_p = jnp.concatenate([src_p, la_src], axis=1)
  dst_p = jnp.concatenate([dst_p, la_dst], axis=1)

  x_pad = jnp.zeros((npad, d_in), x.dtype).at[:n].set(x)

  degp = _sc_degree(dst_p, npad)

  m1 = _tc_matmul(x_pad, W1, blk)
  g1, dinv = _tc_scale(m1, degp.reshape(nc, npad, 1), blk)

  idx_c = jnp.stack([src_p, dst_p], axis=2)
  p1 = _sc_segsum_rows(g1, idx_c, npad, d_hid, nch)

  g2 = _tc_layer2_in(p1, g1, dinv, b1.reshape(1, d_hid), W2, blk)

  p2 = _sc_segsum_1d(g2.reshape(npad), src_p, dst_p, npad)

  out = _tc_combine(p2.reshape(nc, npad, 1), g2, dinv,
                    b2.reshape(1, 1), blk)
  return out[:n]


# speedup vs baseline: 1.0320x; 1.0320x over previous
import functools
import math

import jax
import jax.numpy as jnp
from jax import lax
from jax.experimental import pallas as pl
from jax.experimental.pallas import tpu as pltpu
from jax.experimental.pallas import tpu_sc as plsc

CH = 128
LANES = 16


def _sc_mesh_info():
  info = plsc.get_sparse_core_info()
  return info.num_cores, info.num_subcores


def _sc_degree(dst_idx, npad):
  nc, ns = _sc_mesh_info()
  nw, nch, _ = dst_idx.shape
  rows_per_tile = npad // ns
  GRP = 12
  mesh = plsc.VectorSubcoreMesh(core_axis_name="c", subcore_axis_name="s")

  @functools.partial(
      pl.kernel,
      out_type=jax.ShapeDtypeStruct((nc, npad), jnp.float32),
      mesh=mesh,
      scratch_types=[
          pltpu.VMEM((nch, CH), jnp.int32),
          pltpu.VMEM((CH,), jnp.float32),
          pltpu.VMEM((rows_per_tile,), jnp.float32),
          pltpu.VMEM_SHARED((npad,), jnp.float32),
          pltpu.SemaphoreType.DMA,
      ],
  )
  def k(didx_hbm, out_hbm, didx_v, vals_v, z_v, acc_sh, sem):
    c = lax.axis_index("c")
    s = lax.axis_index("s")
    wid = s * nc + c
    pltpu.sync_copy(didx_hbm.at[wid], didx_v)
    for i in range(0, CH, LANES):
      vals_v[pl.ds(i, LANES)] = jnp.ones((LANES,), jnp.float32)
    for i in range(0, rows_per_tile, LANES):
      z_v[pl.ds(i, LANES)] = jnp.zeros((LANES,), jnp.float32)
    row0 = s * rows_per_tile
    pltpu.sync_copy(z_v, acc_sh.at[pl.ds(row0, rows_per_tile)])
    plsc.subcore_barrier()

    def body(g, carry):
      j0 = GRP * g
      for q in range(GRP):
        pltpu.async_copy(vals_v, acc_sh.at[didx_v.at[j0 + q]], sem, add=True)
      for q in range(GRP):
        pltpu.make_async_copy(vals_v, acc_sh.at[didx_v.at[0]], sem).wait()
      return carry

    lax.fori_loop(0, nch // GRP, body, 0)
    for j in range(nch - nch % GRP, nch):
      pltpu.sync_copy(vals_v, acc_sh.at[didx_v.at[j]], add=True)
    plsc.subcore_barrier()
    pltpu.sync_copy(acc_sh.at[pl.ds(row0, rows_per_tile)],
                    out_hbm.at[c, pl.ds(row0, rows_per_tile)])

  return k(dst_idx)


def _sc_segsum_1d(vals, src_idx, dst_idx, npad):
  nc, ns = _sc_mesh_info()
  nw, nch_a, _ = dst_idx.shape
  rows_per_tile = npad // ns
  mesh = plsc.VectorSubcoreMesh(core_axis_name="c", subcore_axis_name="s")

  @functools.partial(
      pl.kernel,
      out_type=jax.ShapeDtypeStruct((nc, npad), jnp.float32),
      mesh=mesh,
      scratch_types=[
          pltpu.VMEM((nch_a, CH), jnp.int32),
          pltpu.VMEM((nch_a, CH), jnp.int32),
          pltpu.VMEM((4, CH), jnp.float32),
          pltpu.VMEM((rows_per_tile,), jnp.float32),
          pltpu.VMEM_SHARED((npad,), jnp.float32),
          pltpu.SemaphoreType.DMA,
          pltpu.SemaphoreType.DMA,
          pltpu.SemaphoreType.DMA,
          pltpu.SemaphoreType.DMA,
      ],
  )
  def k(vals_hbm, sidx_hbm, didx_hbm, out_hbm, sidx_v, didx_v, bufs, z_v,
        acc_sh, *gsems):
    c = lax.axis_index("c")
    s = lax.axis_index("s")
    wid = s * nc + c
    pltpu.sync_copy(sidx_hbm.at[wid], sidx_v)
    pltpu.sync_copy(didx_hbm.at[wid], didx_v)
    for i in range(0, rows_per_tile, LANES):
      z_v[pl.ds(i, LANES)] = jnp.zeros((LANES,), jnp.float32)
    row0 = s * rows_per_tile
    pltpu.sync_copy(z_v, acc_sh.at[pl.ds(row0, rows_per_tile)])
    plsc.subcore_barrier()

    for q in range(4):
      pltpu.async_copy(vals_hbm.at[sidx_v.at[q]], bufs.at[q], gsems[q])

    def body(k4, carry):
      j0 = 4 * k4
      for q in range(4):
        pltpu.make_async_copy(
            vals_hbm.at[sidx_v.at[0]], bufs.at[q], gsems[q]).wait()
        pltpu.sync_copy(bufs.at[q], acc_sh.at[didx_v.at[j0 + q]], add=True)
        nxt = jnp.minimum(j0 + 4 + q, nch_a - 1)
        pltpu.async_copy(vals_hbm.at[sidx_v.at[nxt]], bufs.at[q], gsems[q])
      return carry

    lax.fori_loop(0, nch_a // 4, body, 0)
    for q in range(4):
      pltpu.make_async_copy(
          vals_hbm.at[sidx_v.at[0]], bufs.at[q], gsems[q]).wait()
    plsc.subcore_barrier()
    pltpu.sync_copy(acc_sh.at[pl.ds(row0, rows_per_tile)],
                    out_hbm.at[c, pl.ds(row0, rows_per_tile)])

  return k(vals, src_idx, dst_idx)


def _sc_segsum_rows(table, idx, npad, d, nch):
  nc, ns = _sc_mesh_info()
  nw, nch_a, two, ch = idx.shape
  rows_per_tile = npad // ns
  zrows = 16
  mesh = plsc.VectorSubcoreMesh(core_axis_name="c", subcore_axis_name="s")

  @functools.partial(
      pl.kernel,
      out_type=jax.ShapeDtypeStruct((nc, npad, d), jnp.float32),
      mesh=mesh,
      scratch_types=[
          pltpu.VMEM((4, 2, ch), jnp.int32),
          pltpu.VMEM((ch, d), jnp.float32),
          pltpu.VMEM((ch, d), jnp.float32),
          pltpu.VMEM((zrows, d), jnp.float32),
          pltpu.VMEM_SHARED((npad, d), jnp.float32),
          pltpu.SemaphoreType.DMA,
          pltpu.SemaphoreType.DMA,
          pltpu.SemaphoreType.DMA,
      ],
  )
  def k(tab_hbm, idx_hbm, out_hbm, islot, buf0, buf1, z_v, acc_sh,
        gsem0, gsem1, isem):
    c = lax.axis_index("c")
    s = lax.axis_index("s")
    wid = s * nc + c
    for r in range(zrows):
      for i in range(0, d, LANES):
        z_v[r, pl.ds(i, LANES)] = jnp.zeros((LANES,), jnp.float32)
    row0 = s * rows_per_tile

    def zbody(t, carry):
      pltpu.sync_copy(z_v, acc_sh.at[pl.ds(row0 + t * zrows, zrows)])
      return carry

    lax.fori_loop(0, rows_per_tile // zrows, zbody, 0)
    plsc.subcore_barrier()

    for q in range(4):
      pltpu.sync_copy(idx_hbm.at[wid, q], islot.at[q])
    pltpu.async_copy(tab_hbm.at[islot.at[0, 0]], buf0, gsem0)
    pltpu.async_copy(tab_hbm.at[islot.at[1, 0]], buf1, gsem1)

    def body(k2, carry):
      j0 = 2 * k2
      s0 = lax.rem(j0, 4)
      s1 = lax.rem(j0 + 1, 4)
      s2 = lax.rem(j0 + 2, 4)
      s3 = lax.rem(j0 + 3, 4)
      pltpu.make_async_copy(tab_hbm.at[islot.at[s0, 0]], buf0, gsem0).wait()
      pltpu.sync_copy(buf0, acc_sh.at[islot.at[s0, 1]], add=True)
      pltpu.async_copy(idx_hbm.at[wid, j0 + 4], islot.at[s0], isem)
      pltpu.async_copy(tab_hbm.at[islot.at[s2, 0]], buf0, gsem0)
      pltpu.make_async_copy(tab_hbm.at[islot.at[s1, 0]], buf1, gsem1).wait()
      pltpu.sync_copy(buf1, acc_sh.at[islot.at[s1, 1]], add=True)
      pltpu.async_copy(idx_hbm.at[wid, j0 + 5], islot.at[s1], isem)
      pltpu.async_copy(tab_hbm.at[islot.at[s3, 0]], buf1, gsem1)
      pltpu.make_async_copy(idx_hbm.at[wid, 0], islot.at[s0], isem).wait()
      pltpu.make_async_copy(idx_hbm.at[wid, 0], islot.at[s1], isem).wait()
      return carry

    lax.fori_loop(0, nch // 2, body, 0)
    pltpu.make_async_copy(tab_hbm.at[islot.at[0, 0]], buf0, gsem0).wait()
    pltpu.make_async_copy(tab_hbm.at[islot.at[1, 0]], buf1, gsem1).wait()
    plsc.subcore_barrier()
    pltpu.sync_copy(acc_sh.at[pl.ds(row0, rows_per_tile)],
                    out_hbm.at[c, pl.ds(row0, rows_per_tile)])

  return k(table, idx)


def _sc_segsum_rows_serial(table, src_idx, dst_idx, npad, d):
  nc, ns = _sc_mesh_info()
  nw, nch, _ = dst_idx.shape
  rows_per_tile = npad // ns
  zrows = 16
  mesh = plsc.VectorSubcoreMesh(core_axis_name="c", subcore_axis_name="s")

  @functools.partial(
      pl.kernel,
      out_type=jax.ShapeDtypeStruct((nc, npad, d), jnp.float32),
      mesh=mesh,
      scratch_types=[
          pltpu.VMEM((nch, CH), jnp.int32),
          pltpu.VMEM((nch, CH), jnp.int32),
          pltpu.VMEM((CH, d), jnp.float32),
          pltpu.VMEM((zrows, d), jnp.float32),
          pltpu.VMEM_SHARED((npad, d), jnp.float32),
          pltpu.SemaphoreType.DMA,
      ],
  )
  def k(tab_hbm, sidx_hbm, didx_hbm, out_hbm, sidx_v, didx_v, rows_v, z_v,
        acc_sh, sem):
    c = lax.axis_index("c")
    s = lax.axis_index("s")
    wid = s * nc + c
    pltpu.sync_copy(sidx_hbm.at[wid], sidx_v)
    pltpu.sync_copy(didx_hbm.at[wid], didx_v)
    for r in range(zrows):
      for i in range(0, d, LANES):
        z_v[r, pl.ds(i, LANES)] = jnp.zeros((LANES,), jnp.float32)
    row0 = s * rows_per_tile

    def zbody(t, carry):
      pltpu.sync_copy(z_v, acc_sh.at[pl.ds(row0 + t * zrows, zrows)])
      return carry

    lax.fori_loop(0, rows_per_tile // zrows, zbody, 0)
    plsc.subcore_barrier()

    def body(j, carry):
      pltpu.async_copy(tab_hbm.at[sidx_v.at[j]], rows_v, sem).wait()
      pltpu.sync_copy(rows_v, acc_sh.at[didx_v.at[j]], add=True)
      return carry

    lax.fori_loop(0, nch, body, 0)
    plsc.subcore_barrier()
    pltpu.sync_copy(acc_sh.at[pl.ds(row0, rows_per_tile)],
                    out_hbm.at[c, pl.ds(row0, rows_per_tile)])

  return k(table, src_idx, dst_idx)


def _tc_matmul_scale(x_pad, w1, degp, blk):
  npad, d_in = x_pad.shape
  d_hid = w1.shape[1]
  nc = degp.shape[0]
  grid = npad // blk

  def body(xb, wb, degb, g1b, dinvb):
    deg = degb[0] + degb[1] + 1.0
    dinv = lax.rsqrt(deg)
    mm = jnp.dot(xb[...], wb[...], preferred_element_type=jnp.float32)
    g1b[...] = mm * dinv
    dinvb[...] = dinv

  return pl.pallas_call(
      body,
      grid=(grid,),
      in_specs=[
          pl.BlockSpec((blk, d_in), lambda i: (i, 0)),
          pl.BlockSpec((d_in, d_hid), lambda i: (0, 0)),
          pl.BlockSpec((nc, blk, 1), lambda i: (0, i, 0)),
      ],
      out_specs=[
          pl.BlockSpec((blk, d_hid), lambda i: (i, 0)),
          pl.BlockSpec((blk, 1), lambda i: (i, 0)),
      ],
      out_shape=[
          jax.ShapeDtypeStruct((npad, d_hid), jnp.float32),
          jax.ShapeDtypeStruct((npad, 1), jnp.float32),
      ],
  )(x_pad, w1, degp)


def _tc_layer2_in(p1, g1, dinv, b1, w2, blk):
  nc, npad, d_hid = p1.shape
  d_out = w2.shape[1]
  grid = npad // blk

  def body(pb, g1b, dinvb, b1b, wb, g2b):
    s = (pb[0] + pb[1] + g1b[...]) * dinvb[...]
    h = jnp.maximum(s + b1b[...], 0.0)
    mm = jnp.dot(h, wb[...], preferred_element_type=jnp.float32)
    g2b[...] = mm * dinvb[...]

  return pl.pallas_call(
      body,
      grid=(grid,),
      in_specs=[
          pl.BlockSpec((nc, blk, d_hid), lambda i: (0, i, 0)),
          pl.BlockSpec((blk, d_hid), lambda i: (i, 0)),
          pl.BlockSpec((blk, 1), lambda i: (i, 0)),
          pl.BlockSpec((1, d_hid), lambda i: (0, 0)),
          pl.BlockSpec((d_hid, d_out), lambda i: (0, 0)),
      ],
      out_specs=pl.BlockSpec((blk, d_out), lambda i: (i, 0)),
      out_shape=jax.ShapeDtypeStruct((npad, d_out), jnp.float32),
  )(p1, g1, dinv, b1, w2)


def _tc_combine(p2, g2, dinv, b2, blk):
  nc, npad, d_out = p2.shape
  grid = npad // blk

  def body(pb, g2b, dinvb, b2b, outb):
    outb[...] = (pb[0] + pb[1] + g2b[...]) * dinvb[...] + b2b[...]

  return pl.pallas_call(
      body,
      grid=(grid,),
      in_specs=[
          pl.BlockSpec((nc, blk, d_out), lambda i: (0, i, 0)),
          pl.BlockSpec((blk, d_out), lambda i: (i, 0)),
          pl.BlockSpec((blk, 1), lambda i: (i, 0)),
          pl.BlockSpec((1, 1), lambda i: (0, 0)),
      ],
      out_specs=pl.BlockSpec((blk, d_out), lambda i: (i, 0)),
      out_shape=jax.ShapeDtypeStruct((npad, d_out), jnp.float32),
  )(p2, g2, dinv, b2)


def kernel(x, edge_index, W1, b1, W2, b2):
  n, d_in = x.shape
  d_hid = W1.shape[1]
  d_out = W2.shape[1]
  e = edge_index.shape[1]
  nc, ns = _sc_mesh_info()
  nw = nc * ns

  blk = 1024
  npad = ((n + 1 + blk - 1) // blk) * blk
  dump = n

  nch = math.ceil(e / (nw * CH))
  nch = ((nch + 3) // 4) * 4
  e_pad = nw * nch * CH
  src = edge_index[0].astype(jnp.int32)
  dst = edge_index[1].astype(jnp.int32)
  pad_n = e_pad - e
  pad_src = jnp.arange(pad_n, dtype=jnp.int32) % n
  pad_dst = n + jnp.arange(pad_n, dtype=jnp.int32) % (npad - n)
  la_n = nw * 4 * CH
  la_src = (jnp.arange(la_n, dtype=jnp.int32) % n).reshape(nw, 4, CH)
  la_dst = (n + jnp.arange(la_n, dtype=jnp.int32) % (npad - n)).reshape(
      nw, 4, CH)
  src_p = jnp.concatenate([src, pad_src]).reshape(nw, nch, CH)
  dst_p = jnp.concatenate([dst, pad_dst]).reshape(nw, nch, CH)
  src_p = jnp.concatenate([src_p, la_src], axis=1)
  dst_p = jnp.concatenate([dst_p, la_dst], axis=1)

  x_pad = jnp.zeros((npad, d_in), x.dtype).at[:n].set(x)

  degp = _sc_degree(dst_p, npad)

  g1, dinv = _tc_matmul_scale(x_pad, W1, degp.reshape(nc, npad, 1), blk)

  idx_c = jnp.stack([src_p, dst_p], axis=2)
  p1 = _sc_segsum_rows(g1, idx_c, npad, d_hid, nch)

  g2 = _tc_layer2_in(p1, g1, dinv, b1.reshape(1, d_hid), W2, blk)

  p2 = _sc_segsum_1d(g2.reshape(npad), src_p, dst_p, npad)

  out = _tc_combine(p2.reshape(nc, npad, 1), g2, dinv,
                    b2.reshape(1, 1), blk)
  return out[:n]


# speedup vs baseline: 1.0448x; 1.0125x over previous
import functools
import math

import jax
import jax.numpy as jnp
from jax import lax
from jax.experimental import pallas as pl
from jax.experimental.pallas import tpu as pltpu
from jax.experimental.pallas import tpu_sc as plsc

CH = 128
LANES = 16


def _sc_mesh_info():
  info = plsc.get_sparse_core_info()
  return info.num_cores, info.num_subcores


def _sc_degree(dst_idx, npad):
  nc, ns = _sc_mesh_info()
  nw, nch, _ = dst_idx.shape
  rows_per_tile = npad // ns
  GRP = 12
  mesh = plsc.VectorSubcoreMesh(core_axis_name="c", subcore_axis_name="s")

  @functools.partial(
      pl.kernel,
      out_type=jax.ShapeDtypeStruct((nc, npad), jnp.float32),
      mesh=mesh,
      scratch_types=[
          pltpu.VMEM((nch, CH), jnp.int32),
          pltpu.VMEM((CH,), jnp.float32),
          pltpu.VMEM((rows_per_tile,), jnp.float32),
          pltpu.VMEM_SHARED((npad,), jnp.float32),
          pltpu.SemaphoreType.DMA,
      ],
  )
  def k(didx_hbm, out_hbm, didx_v, vals_v, z_v, acc_sh, sem):
    c = lax.axis_index("c")
    s = lax.axis_index("s")
    wid = s * nc + c
    pltpu.sync_copy(didx_hbm.at[wid], didx_v)
    for i in range(0, CH, LANES):
      vals_v[pl.ds(i, LANES)] = jnp.ones((LANES,), jnp.float32)
    for i in range(0, rows_per_tile, LANES):
      z_v[pl.ds(i, LANES)] = jnp.zeros((LANES,), jnp.float32)
    row0 = s * rows_per_tile
    pltpu.sync_copy(z_v, acc_sh.at[pl.ds(row0, rows_per_tile)])
    plsc.subcore_barrier()

    def body(g, carry):
      j0 = GRP * g
      for q in range(GRP):
        pltpu.async_copy(vals_v, acc_sh.at[didx_v.at[j0 + q]], sem, add=True)
      for q in range(GRP):
        pltpu.make_async_copy(vals_v, acc_sh.at[didx_v.at[0]], sem).wait()
      return carry

    lax.fori_loop(0, nch // GRP, body, 0)
    for j in range(nch - nch % GRP, nch):
      pltpu.sync_copy(vals_v, acc_sh.at[didx_v.at[j]], add=True)
    plsc.subcore_barrier()
    pltpu.sync_copy(acc_sh.at[pl.ds(row0, rows_per_tile)],
                    out_hbm.at[c, pl.ds(row0, rows_per_tile)])

  return k(dst_idx)


def _sc_segsum_1d(vals, src_idx, dst_idx, npad):
  nc, ns = _sc_mesh_info()
  nw, nch_a, _ = dst_idx.shape
  rows_per_tile = npad // ns
  DEP = 8
  nfull = (nch_a // DEP) * DEP
  mesh = plsc.VectorSubcoreMesh(core_axis_name="c", subcore_axis_name="s")

  @functools.partial(
      pl.kernel,
      out_type=jax.ShapeDtypeStruct((nc, npad), jnp.float32),
      mesh=mesh,
      scratch_types=[
          pltpu.VMEM((nch_a, CH), jnp.int32),
          pltpu.VMEM((nch_a, CH), jnp.int32),
          pltpu.VMEM((DEP, CH), jnp.float32),
          pltpu.VMEM((rows_per_tile,), jnp.float32),
          pltpu.VMEM_SHARED((npad,), jnp.float32),
      ] + [pltpu.SemaphoreType.DMA] * DEP,
  )
  def k(vals_hbm, sidx_hbm, didx_hbm, out_hbm, sidx_v, didx_v, bufs, z_v,
        acc_sh, *gsems):
    c = lax.axis_index("c")
    s = lax.axis_index("s")
    wid = s * nc + c
    pltpu.sync_copy(sidx_hbm.at[wid], sidx_v)
    pltpu.sync_copy(didx_hbm.at[wid], didx_v)
    for i in range(0, rows_per_tile, LANES):
      z_v[pl.ds(i, LANES)] = jnp.zeros((LANES,), jnp.float32)
    row0 = s * rows_per_tile
    pltpu.sync_copy(z_v, acc_sh.at[pl.ds(row0, rows_per_tile)])
    plsc.subcore_barrier()

    for q in range(DEP):
      pltpu.async_copy(vals_hbm.at[sidx_v.at[q]], bufs.at[q], gsems[q])

    def body(kk, carry):
      j0 = DEP * kk
      for q in range(DEP):
        pltpu.make_async_copy(
            vals_hbm.at[sidx_v.at[0]], bufs.at[q], gsems[q]).wait()
        pltpu.sync_copy(bufs.at[q], acc_sh.at[didx_v.at[j0 + q]], add=True)
        nxt = jnp.minimum(j0 + DEP + q, nch_a - 1)
        pltpu.async_copy(vals_hbm.at[sidx_v.at[nxt]], bufs.at[q], gsems[q])
      return carry

    lax.fori_loop(0, nch_a // DEP, body, 0)
    for q in range(DEP):
      pltpu.make_async_copy(
          vals_hbm.at[sidx_v.at[0]], bufs.at[q], gsems[q]).wait()
      if nfull + q < nch_a:
        pltpu.sync_copy(bufs.at[q], acc_sh.at[didx_v.at[nfull + q]],
                        add=True)
    plsc.subcore_barrier()
    pltpu.sync_copy(acc_sh.at[pl.ds(row0, rows_per_tile)],
                    out_hbm.at[c, pl.ds(row0, rows_per_tile)])

  return k(vals, src_idx, dst_idx)


def _sc_segsum_rows(table, idx, npad, d, nch):
  nc, ns = _sc_mesh_info()
  nw, nch_a, two, ch = idx.shape
  rows_per_tile = npad // ns
  zrows = 16
  mesh = plsc.VectorSubcoreMesh(core_axis_name="c", subcore_axis_name="s")

  @functools.partial(
      pl.kernel,
      out_type=jax.ShapeDtypeStruct((nc, npad, d), jnp.float32),
      mesh=mesh,
      scratch_types=[
          pltpu.VMEM((4, 2, ch), jnp.int32),
          pltpu.VMEM((ch, d), jnp.float32),
          pltpu.VMEM((ch, d), jnp.float32),
          pltpu.VMEM((zrows, d), jnp.float32),
          pltpu.VMEM_SHARED((npad, d), jnp.float32),
          pltpu.SemaphoreType.DMA,
          pltpu.SemaphoreType.DMA,
          pltpu.SemaphoreType.DMA,
      ],
  )
  def k(tab_hbm, idx_hbm, out_hbm, islot, buf0, buf1, z_v, acc_sh,
        gsem0, gsem1, isem):
    c = lax.axis_index("c")
    s = lax.axis_index("s")
    wid = s * nc + c
    for r in range(zrows):
      for i in range(0, d, LANES):
        z_v[r, pl.ds(i, LANES)] = jnp.zeros((LANES,), jnp.float32)
    row0 = s * rows_per_tile

    def zbody(t, carry):
      pltpu.sync_copy(z_v, acc_sh.at[pl.ds(row0 + t * zrows, zrows)])
      return carry

    lax.fori_loop(0, rows_per_tile // zrows, zbody, 0)
    plsc.subcore_barrier()

    for q in range(4):
      pltpu.sync_copy(idx_hbm.at[wid, q], islot.at[q])
    pltpu.async_copy(tab_hbm.at[islot.at[0, 0]], buf0, gsem0)
    pltpu.async_copy(tab_hbm.at[islot.at[1, 0]], buf1, gsem1)

    def body(k2, carry):
      j0 = 2 * k2
      s0 = lax.rem(j0, 4)
      s1 = lax.rem(j0 + 1, 4)
      s2 = lax.rem(j0 + 2, 4)
      s3 = lax.rem(j0 + 3, 4)
      pltpu.make_async_copy(tab_hbm.at[islot.at[s0, 0]], buf0, gsem0).wait()
      pltpu.sync_copy(buf0, acc_sh.at[islot.at[s0, 1]], add=True)
      pltpu.async_copy(idx_hbm.at[wid, j0 + 4], islot.at[s0], isem)
      pltpu.async_copy(tab_hbm.at[islot.at[s2, 0]], buf0, gsem0)
      pltpu.make_async_copy(tab_hbm.at[islot.at[s1, 0]], buf1, gsem1).wait()
      pltpu.sync_copy(buf1, acc_sh.at[islot.at[s1, 1]], add=True)
      pltpu.async_copy(idx_hbm.at[wid, j0 + 5], islot.at[s1], isem)
      pltpu.async_copy(tab_hbm.at[islot.at[s3, 0]], buf1, gsem1)
      pltpu.make_async_copy(idx_hbm.at[wid, 0], islot.at[s0], isem).wait()
      pltpu.make_async_copy(idx_hbm.at[wid, 0], islot.at[s1], isem).wait()
      return carry

    lax.fori_loop(0, nch // 2, body, 0)
    pltpu.make_async_copy(tab_hbm.at[islot.at[0, 0]], buf0, gsem0).wait()
    pltpu.make_async_copy(tab_hbm.at[islot.at[1, 0]], buf1, gsem1).wait()
    plsc.subcore_barrier()
    pltpu.sync_copy(acc_sh.at[pl.ds(row0, rows_per_tile)],
                    out_hbm.at[c, pl.ds(row0, rows_per_tile)])

  return k(table, idx)


def _sc_segsum_rows_serial(table, src_idx, dst_idx, npad, d):
  nc, ns = _sc_mesh_info()
  nw, nch, _ = dst_idx.shape
  rows_per_tile = npad // ns
  zrows = 16
  mesh = plsc.VectorSubcoreMesh(core_axis_name="c", subcore_axis_name="s")

  @functools.partial(
      pl.kernel,
      out_type=jax.ShapeDtypeStruct((nc, npad, d), jnp.float32),
      mesh=mesh,
      scratch_types=[
          pltpu.VMEM((nch, CH), jnp.int32),
          pltpu.VMEM((nch, CH), jnp.int32),
          pltpu.VMEM((CH, d), jnp.float32),
          pltpu.VMEM((zrows, d), jnp.float32),
          pltpu.VMEM_SHARED((npad, d), jnp.float32),
          pltpu.SemaphoreType.DMA,
      ],
  )
  def k(tab_hbm, sidx_hbm, didx_hbm, out_hbm, sidx_v, didx_v, rows_v, z_v,
        acc_sh, sem):
    c = lax.axis_index("c")
    s = lax.axis_index("s")
    wid = s * nc + c
    pltpu.sync_copy(sidx_hbm.at[wid], sidx_v)
    pltpu.sync_copy(didx_hbm.at[wid], didx_v)
    for r in range(zrows):
      for i in range(0, d, LANES):
        z_v[r, pl.ds(i, LANES)] = jnp.zeros((LANES,), jnp.float32)
    row0 = s * rows_per_tile

    def zbody(t, carry):
      pltpu.sync_copy(z_v, acc_sh.at[pl.ds(row0 + t * zrows, zrows)])
      return carry

    lax.fori_loop(0, rows_per_tile // zrows, zbody, 0)
    plsc.subcore_barrier()

    def body(j, carry):
      pltpu.async_copy(tab_hbm.at[sidx_v.at[j]], rows_v, sem).wait()
      pltpu.sync_copy(rows_v, acc_sh.at[didx_v.at[j]], add=True)
      return carry

    lax.fori_loop(0, nch, body, 0)
    plsc.subcore_barrier()
    pltpu.sync_copy(acc_sh.at[pl.ds(row0, rows_per_tile)],
                    out_hbm.at[c, pl.ds(row0, rows_per_tile)])

  return k(table, src_idx, dst_idx)


def _tc_matmul_scale(x_pad, w1, degp, blk):
  npad, d_in = x_pad.shape
  d_hid = w1.shape[1]
  nc = degp.shape[0]
  grid = npad // blk

  def body(xb, wb, degb, g1b, dinvb):
    deg = degb[0] + degb[1] + 1.0
    dinv = lax.rsqrt(deg)
    mm = jnp.dot(xb[...], wb[...], preferred_element_type=jnp.float32)
    g1b[...] = mm * dinv
    dinvb[...] = dinv

  return pl.pallas_call(
      body,
      grid=(grid,),
      in_specs=[
          pl.BlockSpec((blk, d_in), lambda i: (i, 0)),
          pl.BlockSpec((d_in, d_hid), lambda i: (0, 0)),
          pl.BlockSpec((nc, blk, 1), lambda i: (0, i, 0)),
      ],
      out_specs=[
          pl.BlockSpec((blk, d_hid), lambda i: (i, 0)),
          pl.BlockSpec((blk, 1), lambda i: (i, 0)),
      ],
      out_shape=[
          jax.ShapeDtypeStruct((npad, d_hid), jnp.float32),
          jax.ShapeDtypeStruct((npad, 1), jnp.float32),
      ],
  )(x_pad, w1, degp)


def _tc_layer2_in(p1, g1, dinv, b1, w2, blk):
  nc, npad, d_hid = p1.shape
  d_out = w2.shape[1]
  grid = npad // blk

  def body(pb, g1b, dinvb, b1b, wb, g2b):
    s = (pb[0] + pb[1] + g1b[...]) * dinvb[...]
    h = jnp.maximum(s + b1b[...], 0.0)
    mm = jnp.dot(h, wb[...], preferred_element_type=jnp.float32)
    g2b[...] = mm * dinvb[...]

  return pl.pallas_call(
      body,
      grid=(grid,),
      in_specs=[
          pl.BlockSpec((nc, blk, d_hid), lambda i: (0, i, 0)),
          pl.BlockSpec((blk, d_hid), lambda i: (i, 0)),
          pl.BlockSpec((blk, 1), lambda i: (i, 0)),
          pl.BlockSpec((1, d_hid), lambda i: (0, 0)),
          pl.BlockSpec((d_hid, d_out), lambda i: (0, 0)),
      ],
      out_specs=pl.BlockSpec((blk, d_out), lambda i: (i, 0)),
      out_shape=jax.ShapeDtypeStruct((npad, d_out), jnp.float32),
  )(p1, g1, dinv, b1, w2)


def _tc_combine(p2, g2, dinv, b2, blk):
  nc, npad, d_out = p2.shape
  grid = npad // blk

  def body(pb, g2b, dinvb, b2b, outb):
    outb[...] = (pb[0] + pb[1] + g2b[...]) * dinvb[...] + b2b[...]

  return pl.pallas_call(
      body,
      grid=(grid,),
      in_specs=[
          pl.BlockSpec((nc, blk, d_out), lambda i: (0, i, 0)),
          pl.BlockSpec((blk, d_out), lambda i: (i, 0)),
          pl.BlockSpec((blk, 1), lambda i: (i, 0)),
          pl.BlockSpec((1, 1), lambda i: (0, 0)),
      ],
      out_specs=pl.BlockSpec((blk, d_out), lambda i: (i, 0)),
      out_shape=jax.ShapeDtypeStruct((npad, d_out), jnp.float32),
  )(p2, g2, dinv, b2)


def kernel(x, edge_index, W1, b1, W2, b2):
  n, d_in = x.shape
  d_hid = W1.shape[1]
  d_out = W2.shape[1]
  e = edge_index.shape[1]
  nc, ns = _sc_mesh_info()
  nw = nc * ns

  blk = 1024
  npad = ((n + 1 + blk - 1) // blk) * blk
  dump = n

  nch = math.ceil(e / (nw * CH))
  nch = ((nch + 3) // 4) * 4
  e_pad = nw * nch * CH
  src = edge_index[0].astype(jnp.int32)
  dst = edge_index[1].astype(jnp.int32)
  pad_n = e_pad - e
  pad_src = jnp.arange(pad_n, dtype=jnp.int32) % n
  pad_dst = n + jnp.arange(pad_n, dtype=jnp.int32) % (npad - n)
  la_n = nw * 4 * CH
  la_src = (jnp.arange(la_n, dtype=jnp.int32) % n).reshape(nw, 4, CH)
  la_dst = (n + jnp.arange(la_n, dtype=jnp.int32) % (npad - n)).reshape(
      nw, 4, CH)
  src_p = jnp.concatenate([src, pad_src]).reshape(nw, nch, CH)
  dst_p = jnp.concatenate([dst, pad_dst]).reshape(nw, nch, CH)
  src_p = jnp.concatenate([src_p, la_src], axis=1)
  dst_p = jnp.concatenate([dst_p, la_dst], axis=1)

  x_pad = jnp.zeros((npad, d_in), x.dtype).at[:n].set(x)

  degp = _sc_degree(dst_p, npad)

  g1, dinv = _tc_matmul_scale(x_pad, W1, degp.reshape(nc, npad, 1), blk)

  idx_c = jnp.stack([src_p, dst_p], axis=2)
  p1 = _sc_segsum_rows(g1, idx_c, npad, d_hid, nch)

  g2 = _tc_layer2_in(p1, g1, dinv, b1.reshape(1, d_hid), W2, blk)

  p2 = _sc_segsum_1d(g2.reshape(npad), src_p, dst_p, npad)

  out = _tc_combine(p2.reshape(nc, npad, 1), g2, dinv,
                    b2.reshape(1, 1), blk)
  return out[:n]


# speedup vs baseline: 1.0653x; 1.0196x over previous
import functools
import math

import jax
import jax.numpy as jnp
from jax import lax
from jax.experimental import pallas as pl
from jax.experimental.pallas import tpu as pltpu
from jax.experimental.pallas import tpu_sc as plsc

CH = 128
LANES = 16


def _sc_mesh_info():
  info = plsc.get_sparse_core_info()
  return info.num_cores, info.num_subcores


def _sc_degree(dst_idx, npad):
  nc, ns = _sc_mesh_info()
  nw, nch, _ = dst_idx.shape
  rows_per_tile = npad // ns
  GRP = 12
  mesh = plsc.VectorSubcoreMesh(core_axis_name="c", subcore_axis_name="s")

  @functools.partial(
      pl.kernel,
      out_type=jax.ShapeDtypeStruct((nc, npad), jnp.float32),
      mesh=mesh,
      scratch_types=[
          pltpu.VMEM((nch, CH), jnp.int32),
          pltpu.VMEM((CH,), jnp.float32),
          pltpu.VMEM((rows_per_tile,), jnp.float32),
          pltpu.VMEM_SHARED((npad,), jnp.float32),
          pltpu.SemaphoreType.DMA,
      ],
  )
  def k(didx_hbm, out_hbm, didx_v, vals_v, z_v, acc_sh, sem):
    c = lax.axis_index("c")
    s = lax.axis_index("s")
    wid = s * nc + c
    pltpu.sync_copy(didx_hbm.at[wid], didx_v)
    for i in range(0, CH, LANES):
      vals_v[pl.ds(i, LANES)] = jnp.ones((LANES,), jnp.float32)
    for i in range(0, rows_per_tile, LANES):
      z_v[pl.ds(i, LANES)] = jnp.zeros((LANES,), jnp.float32)
    row0 = s * rows_per_tile
    pltpu.sync_copy(z_v, acc_sh.at[pl.ds(row0, rows_per_tile)])
    plsc.subcore_barrier()

    def body(g, carry):
      j0 = GRP * g
      for q in range(GRP):
        pltpu.async_copy(vals_v, acc_sh.at[didx_v.at[j0 + q]], sem, add=True)
      for q in range(GRP):
        pltpu.make_async_copy(vals_v, acc_sh.at[didx_v.at[0]], sem).wait()
      return carry

    lax.fori_loop(0, nch // GRP, body, 0)
    for j in range(nch - nch % GRP, nch):
      pltpu.sync_copy(vals_v, acc_sh.at[didx_v.at[j]], add=True)
    plsc.subcore_barrier()
    pltpu.sync_copy(acc_sh.at[pl.ds(row0, rows_per_tile)],
                    out_hbm.at[c, pl.ds(row0, rows_per_tile)])

  return k(dst_idx)


def _sc_segsum_1d(vals, src_idx, dst_idx, npad):
  nc, ns = _sc_mesh_info()
  nw, nch_a, _ = dst_idx.shape
  rows_per_tile = npad // ns
  DEP = 8
  nfull = (nch_a // DEP) * DEP
  mesh = plsc.VectorSubcoreMesh(core_axis_name="c", subcore_axis_name="s")

  @functools.partial(
      pl.kernel,
      out_type=jax.ShapeDtypeStruct((nc, npad), jnp.float32),
      mesh=mesh,
      scratch_types=[
          pltpu.VMEM((nch_a, CH), jnp.int32),
          pltpu.VMEM((nch_a, CH), jnp.int32),
          pltpu.VMEM((DEP, CH), jnp.float32),
          pltpu.VMEM((rows_per_tile,), jnp.float32),
          pltpu.VMEM_SHARED((npad,), jnp.float32),
      ] + [pltpu.SemaphoreType.DMA] * DEP,
  )
  def k(vals_hbm, sidx_hbm, didx_hbm, out_hbm, sidx_v, didx_v, bufs, z_v,
        acc_sh, *gsems):
    c = lax.axis_index("c")
    s = lax.axis_index("s")
    wid = s * nc + c
    pltpu.sync_copy(sidx_hbm.at[wid], sidx_v)
    pltpu.sync_copy(didx_hbm.at[wid], didx_v)
    for i in range(0, rows_per_tile, LANES):
      z_v[pl.ds(i, LANES)] = jnp.zeros((LANES,), jnp.float32)
    row0 = s * rows_per_tile
    pltpu.sync_copy(z_v, acc_sh.at[pl.ds(row0, rows_per_tile)])
    plsc.subcore_barrier()

    for q in range(DEP):
      pltpu.async_copy(vals_hbm.at[sidx_v.at[q]], bufs.at[q], gsems[q])

    def body(kk, carry):
      j0 = DEP * kk
      for q in range(DEP):
        pltpu.make_async_copy(
            vals_hbm.at[sidx_v.at[0]], bufs.at[q], gsems[q]).wait()
        pltpu.sync_copy(bufs.at[q], acc_sh.at[didx_v.at[j0 + q]], add=True)
        nxt = jnp.minimum(j0 + DEP + q, nch_a - 1)
        pltpu.async_copy(vals_hbm.at[sidx_v.at[nxt]], bufs.at[q], gsems[q])
      return carry

    lax.fori_loop(0, nch_a // DEP, body, 0)
    for q in range(DEP):
      pltpu.make_async_copy(
          vals_hbm.at[sidx_v.at[0]], bufs.at[q], gsems[q]).wait()
      if nfull + q < nch_a:
        pltpu.sync_copy(bufs.at[q], acc_sh.at[didx_v.at[nfull + q]],
                        add=True)
    plsc.subcore_barrier()
    pltpu.sync_copy(acc_sh.at[pl.ds(row0, rows_per_tile)],
                    out_hbm.at[c, pl.ds(row0, rows_per_tile)])

  return k(vals, src_idx, dst_idx)


def _sc_segsum_rows(table, idx, npad, d, nch):
  nc, ns = _sc_mesh_info()
  nw, nch_a, two, ch = idx.shape
  rows_per_tile = npad // ns
  zrows = 16
  mesh = plsc.VectorSubcoreMesh(core_axis_name="c", subcore_axis_name="s")

  @functools.partial(
      pl.kernel,
      out_type=jax.ShapeDtypeStruct((nc, npad, d), jnp.float32),
      mesh=mesh,
      scratch_types=[
          pltpu.VMEM((4, 2, ch), jnp.int32),
          pltpu.VMEM((ch, d), jnp.float32),
          pltpu.VMEM((ch, d), jnp.float32),
          pltpu.VMEM((zrows, d), jnp.float32),
          pltpu.VMEM_SHARED((npad, d), jnp.float32),
          pltpu.SemaphoreType.DMA,
          pltpu.SemaphoreType.DMA,
          pltpu.SemaphoreType.DMA,
      ],
  )
  def k(tab_hbm, idx_hbm, out_hbm, islot, buf0, buf1, z_v, acc_sh,
        gsem0, gsem1, isem):
    c = lax.axis_index("c")
    s = lax.axis_index("s")
    wid = s * nc + c
    for r in range(zrows):
      for i in range(0, d, LANES):
        z_v[r, pl.ds(i, LANES)] = jnp.zeros((LANES,), jnp.float32)
    row0 = s * rows_per_tile

    def zbody(t, carry):
      pltpu.sync_copy(z_v, acc_sh.at[pl.ds(row0 + t * zrows, zrows)])
      return carry

    lax.fori_loop(0, rows_per_tile // zrows, zbody, 0)
    plsc.subcore_barrier()

    for q in range(4):
      pltpu.sync_copy(idx_hbm.at[wid, q], islot.at[q])
    pltpu.async_copy(tab_hbm.at[islot.at[0, 0]], buf0, gsem0)
    pltpu.async_copy(tab_hbm.at[islot.at[1, 0]], buf1, gsem1)

    def body(k2, carry):
      j0 = 2 * k2
      s0 = lax.rem(j0, 4)
      s1 = lax.rem(j0 + 1, 4)
      s2 = lax.rem(j0 + 2, 4)
      s3 = lax.rem(j0 + 3, 4)
      pltpu.make_async_copy(tab_hbm.at[islot.at[s0, 0]], buf0, gsem0).wait()
      pltpu.sync_copy(buf0, acc_sh.at[islot.at[s0, 1]], add=True)
      pltpu.async_copy(idx_hbm.at[wid, j0 + 4], islot.at[s0], isem)
      pltpu.async_copy(tab_hbm.at[islot.at[s2, 0]], buf0, gsem0)
      pltpu.make_async_copy(tab_hbm.at[islot.at[s1, 0]], buf1, gsem1).wait()
      pltpu.sync_copy(buf1, acc_sh.at[islot.at[s1, 1]], add=True)
      pltpu.async_copy(idx_hbm.at[wid, j0 + 5], islot.at[s1], isem)
      pltpu.async_copy(tab_hbm.at[islot.at[s3, 0]], buf1, gsem1)
      pltpu.make_async_copy(idx_hbm.at[wid, 0], islot.at[s0], isem).wait()
      pltpu.make_async_copy(idx_hbm.at[wid, 0], islot.at[s1], isem).wait()
      return carry

    lax.fori_loop(0, nch // 2, body, 0)
    pltpu.make_async_copy(tab_hbm.at[islot.at[0, 0]], buf0, gsem0).wait()
    pltpu.make_async_copy(tab_hbm.at[islot.at[1, 0]], buf1, gsem1).wait()
    plsc.subcore_barrier()
    pltpu.sync_copy(acc_sh.at[pl.ds(row0, rows_per_tile)],
                    out_hbm.at[c, pl.ds(row0, rows_per_tile)])

  return k(table, idx)


def _sc_segsum_rows_serial(table, src_idx, dst_idx, npad, d):
  nc, ns = _sc_mesh_info()
  nw, nch, _ = dst_idx.shape
  rows_per_tile = npad // ns
  zrows = 16
  mesh = plsc.VectorSubcoreMesh(core_axis_name="c", subcore_axis_name="s")

  @functools.partial(
      pl.kernel,
      out_type=jax.ShapeDtypeStruct((nc, npad, d), jnp.float32),
      mesh=mesh,
      scratch_types=[
          pltpu.VMEM((nch, CH), jnp.int32),
          pltpu.VMEM((nch, CH), jnp.int32),
          pltpu.VMEM((CH, d), jnp.float32),
          pltpu.VMEM((zrows, d), jnp.float32),
          pltpu.VMEM_SHARED((npad, d), jnp.float32),
          pltpu.SemaphoreType.DMA,
      ],
  )
  def k(tab_hbm, sidx_hbm, didx_hbm, out_hbm, sidx_v, didx_v, rows_v, z_v,
        acc_sh, sem):
    c = lax.axis_index("c")
    s = lax.axis_index("s")
    wid = s * nc + c
    pltpu.sync_copy(sidx_hbm.at[wid], sidx_v)
    pltpu.sync_copy(didx_hbm.at[wid], didx_v)
    for r in range(zrows):
      for i in range(0, d, LANES):
        z_v[r, pl.ds(i, LANES)] = jnp.zeros((LANES,), jnp.float32)
    row0 = s * rows_per_tile

    def zbody(t, carry):
      pltpu.sync_copy(z_v, acc_sh.at[pl.ds(row0 + t * zrows, zrows)])
      return carry

    lax.fori_loop(0, rows_per_tile // zrows, zbody, 0)
    plsc.subcore_barrier()

    def body(j, carry):
      pltpu.async_copy(tab_hbm.at[sidx_v.at[j]], rows_v, sem).wait()
      pltpu.sync_copy(rows_v, acc_sh.at[didx_v.at[j]], add=True)
      return carry

    lax.fori_loop(0, nch, body, 0)
    plsc.subcore_barrier()
    pltpu.sync_copy(acc_sh.at[pl.ds(row0, rows_per_tile)],
                    out_hbm.at[c, pl.ds(row0, rows_per_tile)])

  return k(table, src_idx, dst_idx)


def _tc_matmul_scale(x_pad, w1, degp, blk):
  npad, d_in = x_pad.shape
  d_hid = w1.shape[1]
  nc = degp.shape[0]
  grid = npad // blk

  def body(xb, wb, degb, g1b, dinvb):
    deg = degb[0] + degb[1] + 1.0
    dinv = lax.rsqrt(deg)
    mm = jnp.dot(xb[...], wb[...], preferred_element_type=jnp.float32)
    g1b[...] = mm * dinv
    dinvb[...] = dinv

  return pl.pallas_call(
      body,
      grid=(grid,),
      in_specs=[
          pl.BlockSpec((blk, d_in), lambda i: (i, 0)),
          pl.BlockSpec((d_in, d_hid), lambda i: (0, 0)),
          pl.BlockSpec((nc, blk, 1), lambda i: (0, i, 0)),
      ],
      out_specs=[
          pl.BlockSpec((blk, d_hid), lambda i: (i, 0)),
          pl.BlockSpec((blk, 1), lambda i: (i, 0)),
      ],
      out_shape=[
          jax.ShapeDtypeStruct((npad, d_hid), jnp.float32),
          jax.ShapeDtypeStruct((npad, 1), jnp.float32),
      ],
  )(x_pad, w1, degp)


def _tc_layer2_in(p1, g1, dinv, b1, w2, blk):
  nc, npad, d_hid = p1.shape
  d_out = w2.shape[1]
  grid = npad // blk

  def body(pb, g1b, dinvb, b1b, wb, g2b):
    s = (pb[0] + pb[1] + g1b[...]) * dinvb[...]
    h = jnp.maximum(s + b1b[...], 0.0)
    mm = jnp.dot(h, wb[...], preferred_element_type=jnp.float32)
    g2b[...] = mm * dinvb[...]

  return pl.pallas_call(
      body,
      grid=(grid,),
      in_specs=[
          pl.BlockSpec((nc, blk, d_hid), lambda i: (0, i, 0)),
          pl.BlockSpec((blk, d_hid), lambda i: (i, 0)),
          pl.BlockSpec((blk, 1), lambda i: (i, 0)),
          pl.BlockSpec((1, d_hid), lambda i: (0, 0)),
          pl.BlockSpec((d_hid, d_out), lambda i: (0, 0)),
      ],
      out_specs=pl.BlockSpec((blk, d_out), lambda i: (i, 0)),
      out_shape=jax.ShapeDtypeStruct((npad, d_out), jnp.float32),
  )(p1, g1, dinv, b1, w2)


def _tc_combine(p2, g2, dinv, b2, blk):
  nc, npad, d_out = p2.shape
  grid = npad // blk

  def body(pb, g2b, dinvb, b2b, outb):
    outb[...] = (pb[0] + pb[1] + g2b[...]) * dinvb[...] + b2b[...]

  return pl.pallas_call(
      body,
      grid=(grid,),
      in_specs=[
          pl.BlockSpec((nc, blk, d_out), lambda i: (0, i, 0)),
          pl.BlockSpec((blk, d_out), lambda i: (i, 0)),
          pl.BlockSpec((blk, 1), lambda i: (i, 0)),
          pl.BlockSpec((1, 1), lambda i: (0, 0)),
      ],
      out_specs=pl.BlockSpec((blk, d_out), lambda i: (i, 0)),
      out_shape=jax.ShapeDtypeStruct((npad, d_out), jnp.float32),
  )(p2, g2, dinv, b2)


def kernel(x, edge_index, W1, b1, W2, b2):
  n, d_in = x.shape
  d_hid = W1.shape[1]
  d_out = W2.shape[1]
  e = edge_index.shape[1]
  nc, ns = _sc_mesh_info()
  nw = nc * ns

  npad = ((n + 1 + 1023) // 1024) * 1024
  blk = 2048 if npad % 2048 == 0 else 1024
  dump = n

  nch = math.ceil(e / (nw * CH))
  nch = ((nch + 3) // 4) * 4
  e_pad = nw * nch * CH
  src = edge_index[0].astype(jnp.int32)
  dst = edge_index[1].astype(jnp.int32)
  pad_n = e_pad - e
  pad_src = jnp.arange(pad_n, dtype=jnp.int32) % n
  pad_dst = n + jnp.arange(pad_n, dtype=jnp.int32) % (npad - n)
  la_n = nw * 4 * CH
  la_src = (jnp.arange(la_n, dtype=jnp.int32) % n).reshape(nw, 4, CH)
  la_dst = (n + jnp.arange(la_n, dtype=jnp.int32) % (npad - n)).reshape(
      nw, 4, CH)
  src_p = jnp.concatenate([src, pad_src]).reshape(nw, nch, CH)
  dst_p = jnp.concatenate([dst, pad_dst]).reshape(nw, nch, CH)
  src_p = jnp.concatenate([src_p, la_src], axis=1)
  dst_p = jnp.concatenate([dst_p, la_dst], axis=1)

  x_pad = jnp.zeros((npad, d_in), x.dtype).at[:n].set(x)

  degp = _sc_degree(dst_p, npad)

  g1, dinv = _tc_matmul_scale(x_pad, W1, degp.reshape(nc, npad, 1), blk)

  idx_c = jnp.stack([src_p, dst_p], axis=2)
  p1 = _sc_segsum_rows(g1, idx_c, npad, d_hid, nch)

  g2 = _tc_layer2_in(p1, g1, dinv, b1.reshape(1, d_hid), W2, blk)

  p2 = _sc_segsum_1d(g2.reshape(npad), src_p, dst_p, npad)

  out = _tc_combine(p2.reshape(nc, npad, 1), g2, dinv,
                    b2.reshape(1, 1), blk)
  return out[:n]


# speedup vs baseline: 1.0844x; 1.0179x over previous
import functools
import math

import jax
import jax.numpy as jnp
from jax import lax
from jax.experimental import pallas as pl
from jax.experimental.pallas import tpu as pltpu
from jax.experimental.pallas import tpu_sc as plsc

CH = 128
LANES = 16


def _sc_mesh_info():
  info = plsc.get_sparse_core_info()
  return info.num_cores, info.num_subcores


def _sc_degree(dst_idx, npad):
  nc, ns = _sc_mesh_info()
  nw, nch, _ = dst_idx.shape
  rows_per_tile = npad // ns
  GRP = 12
  mesh = plsc.VectorSubcoreMesh(core_axis_name="c", subcore_axis_name="s")

  @functools.partial(
      pl.kernel,
      out_type=jax.ShapeDtypeStruct((nc, npad), jnp.float32),
      mesh=mesh,
      scratch_types=[
          pltpu.VMEM((nch, CH), jnp.int32),
          pltpu.VMEM((CH,), jnp.float32),
          pltpu.VMEM((rows_per_tile,), jnp.float32),
          pltpu.VMEM_SHARED((npad,), jnp.float32),
          pltpu.SemaphoreType.DMA,
      ],
  )
  def k(didx_hbm, out_hbm, didx_v, vals_v, z_v, acc_sh, sem):
    c = lax.axis_index("c")
    s = lax.axis_index("s")
    wid = s * nc + c
    pltpu.sync_copy(didx_hbm.at[wid], didx_v)
    for i in range(0, CH, LANES):
      vals_v[pl.ds(i, LANES)] = jnp.ones((LANES,), jnp.float32)
    for i in range(0, rows_per_tile, LANES):
      z_v[pl.ds(i, LANES)] = jnp.zeros((LANES,), jnp.float32)
    row0 = s * rows_per_tile
    pltpu.sync_copy(z_v, acc_sh.at[pl.ds(row0, rows_per_tile)])
    plsc.subcore_barrier()

    def body(g, carry):
      j0 = GRP * g
      for q in range(GRP):
        pltpu.async_copy(vals_v, acc_sh.at[didx_v.at[j0 + q]], sem, add=True)
      for q in range(GRP):
        pltpu.make_async_copy(vals_v, acc_sh.at[didx_v.at[0]], sem).wait()
      return carry

    lax.fori_loop(0, nch // GRP, body, 0)
    for j in range(nch - nch % GRP, nch):
      pltpu.sync_copy(vals_v, acc_sh.at[didx_v.at[j]], add=True)
    plsc.subcore_barrier()
    pltpu.sync_copy(acc_sh.at[pl.ds(row0, rows_per_tile)],
                    out_hbm.at[c, pl.ds(row0, rows_per_tile)])

  return k(dst_idx)


def _sc_segsum_1d(vals, src_idx, dst_idx, npad):
  nc, ns = _sc_mesh_info()
  nw, nch_a, _ = dst_idx.shape
  rows_per_tile = npad // ns
  DEP = 8
  nfull = (nch_a // DEP) * DEP
  mesh = plsc.VectorSubcoreMesh(core_axis_name="c", subcore_axis_name="s")

  @functools.partial(
      pl.kernel,
      out_type=jax.ShapeDtypeStruct((nc, npad), jnp.float32),
      mesh=mesh,
      scratch_types=[
          pltpu.VMEM((nch_a, CH), jnp.int32),
          pltpu.VMEM((nch_a, CH), jnp.int32),
          pltpu.VMEM((DEP, CH), jnp.float32),
          pltpu.VMEM((rows_per_tile,), jnp.float32),
          pltpu.VMEM_SHARED((npad,), jnp.float32),
      ] + [pltpu.SemaphoreType.DMA] * DEP,
  )
  def k(vals_hbm, sidx_hbm, didx_hbm, out_hbm, sidx_v, didx_v, bufs, z_v,
        acc_sh, *gsems):
    c = lax.axis_index("c")
    s = lax.axis_index("s")
    wid = s * nc + c
    pltpu.sync_copy(sidx_hbm.at[wid], sidx_v)
    pltpu.sync_copy(didx_hbm.at[wid], didx_v)
    for i in range(0, rows_per_tile, LANES):
      z_v[pl.ds(i, LANES)] = jnp.zeros((LANES,), jnp.float32)
    row0 = s * rows_per_tile
    pltpu.sync_copy(z_v, acc_sh.at[pl.ds(row0, rows_per_tile)])
    plsc.subcore_barrier()

    for q in range(DEP):
      pltpu.async_copy(vals_hbm.at[sidx_v.at[q]], bufs.at[q], gsems[q])

    def body(kk, carry):
      j0 = DEP * kk
      for q in range(DEP):
        pltpu.make_async_copy(
            vals_hbm.at[sidx_v.at[0]], bufs.at[q], gsems[q]).wait()
        pltpu.sync_copy(bufs.at[q], acc_sh.at[didx_v.at[j0 + q]], add=True)
        nxt = jnp.minimum(j0 + DEP + q, nch_a - 1)
        pltpu.async_copy(vals_hbm.at[sidx_v.at[nxt]], bufs.at[q], gsems[q])
      return carry

    lax.fori_loop(0, nch_a // DEP, body, 0)
    for q in range(DEP):
      pltpu.make_async_copy(
          vals_hbm.at[sidx_v.at[0]], bufs.at[q], gsems[q]).wait()
      if nfull + q < nch_a:
        pltpu.sync_copy(bufs.at[q], acc_sh.at[didx_v.at[nfull + q]],
                        add=True)
    plsc.subcore_barrier()
    pltpu.sync_copy(acc_sh.at[pl.ds(row0, rows_per_tile)],
                    out_hbm.at[c, pl.ds(row0, rows_per_tile)])

  return k(vals, src_idx, dst_idx)


def _sc_segsum_rows(table, idx, npad, d, nch):
  nc, ns = _sc_mesh_info()
  nw, nch_a, two, ch = idx.shape
  rows_per_tile = npad // ns
  zrows = 16
  mesh = plsc.VectorSubcoreMesh(core_axis_name="c", subcore_axis_name="s")

  @functools.partial(
      pl.kernel,
      out_type=jax.ShapeDtypeStruct((nc, npad, d), jnp.float32),
      mesh=mesh,
      scratch_types=[
          pltpu.VMEM((4, 2, ch), jnp.int32),
          pltpu.VMEM((ch, d), jnp.float32),
          pltpu.VMEM((ch, d), jnp.float32),
          pltpu.VMEM((zrows, d), jnp.float32),
          pltpu.VMEM_SHARED((npad, d), jnp.float32),
          pltpu.SemaphoreType.DMA,
          pltpu.SemaphoreType.DMA,
          pltpu.SemaphoreType.DMA,
      ],
  )
  def k(tab_hbm, idx_hbm, out_hbm, islot, buf0, buf1, z_v, acc_sh,
        gsem0, gsem1, isem):
    c = lax.axis_index("c")
    s = lax.axis_index("s")
    wid = s * nc + c
    for r in range(zrows):
      for i in range(0, d, LANES):
        z_v[r, pl.ds(i, LANES)] = jnp.zeros((LANES,), jnp.float32)
    row0 = s * rows_per_tile

    def zbody(t, carry):
      pltpu.async_copy(z_v, acc_sh.at[pl.ds(row0 + t * zrows, zrows)], isem)
      return carry

    nz = rows_per_tile // zrows
    lax.fori_loop(0, nz, zbody, 0)

    def zdrain(t, carry):
      pltpu.make_async_copy(
          z_v, acc_sh.at[pl.ds(row0, zrows)], isem).wait()
      return carry

    lax.fori_loop(0, nz, zdrain, 0)
    plsc.subcore_barrier()

    for q in range(4):
      pltpu.sync_copy(idx_hbm.at[wid, q], islot.at[q])
    pltpu.async_copy(tab_hbm.at[islot.at[0, 0]], buf0, gsem0)
    pltpu.async_copy(tab_hbm.at[islot.at[1, 0]], buf1, gsem1)

    def body(k2, carry):
      j0 = 2 * k2
      s0 = lax.rem(j0, 4)
      s1 = lax.rem(j0 + 1, 4)
      s2 = lax.rem(j0 + 2, 4)
      s3 = lax.rem(j0 + 3, 4)
      pltpu.make_async_copy(tab_hbm.at[islot.at[s0, 0]], buf0, gsem0).wait()
      pltpu.sync_copy(buf0, acc_sh.at[islot.at[s0, 1]], add=True)
      pltpu.async_copy(idx_hbm.at[wid, j0 + 4], islot.at[s0], isem)
      pltpu.async_copy(tab_hbm.at[islot.at[s2, 0]], buf0, gsem0)
      pltpu.make_async_copy(tab_hbm.at[islot.at[s1, 0]], buf1, gsem1).wait()
      pltpu.sync_copy(buf1, acc_sh.at[islot.at[s1, 1]], add=True)
      pltpu.async_copy(idx_hbm.at[wid, j0 + 5], islot.at[s1], isem)
      pltpu.async_copy(tab_hbm.at[islot.at[s3, 0]], buf1, gsem1)
      pltpu.make_async_copy(idx_hbm.at[wid, 0], islot.at[s0], isem).wait()
      pltpu.make_async_copy(idx_hbm.at[wid, 0], islot.at[s1], isem).wait()
      return carry

    lax.fori_loop(0, nch // 2, body, 0)
    pltpu.make_async_copy(tab_hbm.at[islot.at[0, 0]], buf0, gsem0).wait()
    pltpu.make_async_copy(tab_hbm.at[islot.at[1, 0]], buf1, gsem1).wait()
    plsc.subcore_barrier()
    pltpu.sync_copy(acc_sh.at[pl.ds(row0, rows_per_tile)],
                    out_hbm.at[c, pl.ds(row0, rows_per_tile)])

  return k(table, idx)


def _sc_segsum_rows_serial(table, src_idx, dst_idx, npad, d):
  nc, ns = _sc_mesh_info()
  nw, nch, _ = dst_idx.shape
  rows_per_tile = npad // ns
  zrows = 16
  mesh = plsc.VectorSubcoreMesh(core_axis_name="c", subcore_axis_name="s")

  @functools.partial(
      pl.kernel,
      out_type=jax.ShapeDtypeStruct((nc, npad, d), jnp.float32),
      mesh=mesh,
      scratch_types=[
          pltpu.VMEM((nch, CH), jnp.int32),
          pltpu.VMEM((nch, CH), jnp.int32),
          pltpu.VMEM((CH, d), jnp.float32),
          pltpu.VMEM((zrows, d), jnp.float32),
          pltpu.VMEM_SHARED((npad, d), jnp.float32),
          pltpu.SemaphoreType.DMA,
      ],
  )
  def k(tab_hbm, sidx_hbm, didx_hbm, out_hbm, sidx_v, didx_v, rows_v, z_v,
        acc_sh, sem):
    c = lax.axis_index("c")
    s = lax.axis_index("s")
    wid = s * nc + c
    pltpu.sync_copy(sidx_hbm.at[wid], sidx_v)
    pltpu.sync_copy(didx_hbm.at[wid], didx_v)
    for r in range(zrows):
      for i in range(0, d, LANES):
        z_v[r, pl.ds(i, LANES)] = jnp.zeros((LANES,), jnp.float32)
    row0 = s * rows_per_tile

    def zbody(t, carry):
      pltpu.sync_copy(z_v, acc_sh.at[pl.ds(row0 + t * zrows, zrows)])
      return carry

    lax.fori_loop(0, rows_per_tile // zrows, zbody, 0)
    plsc.subcore_barrier()

    def body(j, carry):
      pltpu.async_copy(tab_hbm.at[sidx_v.at[j]], rows_v, sem).wait()
      pltpu.sync_copy(rows_v, acc_sh.at[didx_v.at[j]], add=True)
      return carry

    lax.fori_loop(0, nch, body, 0)
    plsc.subcore_barrier()
    pltpu.sync_copy(acc_sh.at[pl.ds(row0, rows_per_tile)],
                    out_hbm.at[c, pl.ds(row0, rows_per_tile)])

  return k(table, src_idx, dst_idx)


def _tc_matmul_scale(x_pad, w1, degp, blk):
  npad, d_in = x_pad.shape
  d_hid = w1.shape[1]
  nc = degp.shape[0]
  grid = npad // blk

  def body(xb, wb, degb, g1b, dinvb):
    deg = degb[0] + degb[1] + 1.0
    dinv = lax.rsqrt(deg)
    mm = jnp.dot(xb[...], wb[...], preferred_element_type=jnp.float32)
    g1b[...] = mm * dinv
    dinvb[...] = dinv

  return pl.pallas_call(
      body,
      grid=(grid,),
      in_specs=[
          pl.BlockSpec((blk, d_in), lambda i: (i, 0)),
          pl.BlockSpec((d_in, d_hid), lambda i: (0, 0)),
          pl.BlockSpec((nc, blk, 1), lambda i: (0, i, 0)),
      ],
      out_specs=[
          pl.BlockSpec((blk, d_hid), lambda i: (i, 0)),
          pl.BlockSpec((blk, 1), lambda i: (i, 0)),
      ],
      out_shape=[
          jax.ShapeDtypeStruct((npad, d_hid), jnp.float32),
          jax.ShapeDtypeStruct((npad, 1), jnp.float32),
      ],
  )(x_pad, w1, degp)


def _tc_layer2_in(p1, g1, dinv, b1, w2, blk):
  nc, npad, d_hid = p1.shape
  d_out = w2.shape[1]
  grid = npad // blk

  def body(pb, g1b, dinvb, b1b, wb, g2b):
    s = (pb[0] + pb[1] + g1b[...]) * dinvb[...]
    h = jnp.maximum(s + b1b[...], 0.0)
    mm = jnp.dot(h, wb[...], preferred_element_type=jnp.float32)
    g2b[...] = mm * dinvb[...]

  return pl.pallas_call(
      body,
      grid=(grid,),
      in_specs=[
          pl.BlockSpec((nc, blk, d_hid), lambda i: (0, i, 0)),
          pl.BlockSpec((blk, d_hid), lambda i: (i, 0)),
          pl.BlockSpec((blk, 1), lambda i: (i, 0)),
          pl.BlockSpec((1, d_hid), lambda i: (0, 0)),
          pl.BlockSpec((d_hid, d_out), lambda i: (0, 0)),
      ],
      out_specs=pl.BlockSpec((blk, d_out), lambda i: (i, 0)),
      out_shape=jax.ShapeDtypeStruct((npad, d_out), jnp.float32),
  )(p1, g1, dinv, b1, w2)


def _tc_combine(p2, g2, dinv, b2, blk):
  nc, npad, d_out = p2.shape
  grid = npad // blk

  def body(pb, g2b, dinvb, b2b, outb):
    outb[...] = (pb[0] + pb[1] + g2b[...]) * dinvb[...] + b2b[...]

  return pl.pallas_call(
      body,
      grid=(grid,),
      in_specs=[
          pl.BlockSpec((nc, blk, d_out), lambda i: (0, i, 0)),
          pl.BlockSpec((blk, d_out), lambda i: (i, 0)),
          pl.BlockSpec((blk, 1), lambda i: (i, 0)),
          pl.BlockSpec((1, 1), lambda i: (0, 0)),
      ],
      out_specs=pl.BlockSpec((blk, d_out), lambda i: (i, 0)),
      out_shape=jax.ShapeDtypeStruct((npad, d_out), jnp.float32),
  )(p2, g2, dinv, b2)


def kernel(x, edge_index, W1, b1, W2, b2):
  n, d_in = x.shape
  d_hid = W1.shape[1]
  d_out = W2.shape[1]
  e = edge_index.shape[1]
  nc, ns = _sc_mesh_info()
  nw = nc * ns

  npad = ((n + 1 + 1023) // 1024) * 1024
  blk = 5120 if npad % 5120 == 0 else 1024
  dump = n

  nch = math.ceil(e / (nw * CH))
  nch = ((nch + 3) // 4) * 4
  e_pad = nw * nch * CH
  src = edge_index[0].astype(jnp.int32)
  dst = edge_index[1].astype(jnp.int32)
  pad_n = e_pad - e
  pad_src = jnp.arange(pad_n, dtype=jnp.int32) % n
  pad_dst = n + jnp.arange(pad_n, dtype=jnp.int32) % (npad - n)
  la_n = nw * 4 * CH
  la_src = (jnp.arange(la_n, dtype=jnp.int32) % n).reshape(nw, 4, CH)
  la_dst = (n + jnp.arange(la_n, dtype=jnp.int32) % (npad - n)).reshape(
      nw, 4, CH)
  src_p = jnp.concatenate([src, pad_src]).reshape(nw, nch, CH)
  dst_p = jnp.concatenate([dst, pad_dst]).reshape(nw, nch, CH)
  src_p = jnp.concatenate([src_p, la_src], axis=1)
  dst_p = jnp.concatenate([dst_p, la_dst], axis=1)

  x_pad = jnp.zeros((npad, d_in), x.dtype).at[:n].set(x)

  degp = _sc_degree(dst_p, npad)

  g1, dinv = _tc_matmul_scale(x_pad, W1, degp.reshape(nc, npad, 1), blk)

  idx_c = jnp.stack([src_p, dst_p], axis=2)
  p1 = _sc_segsum_rows(g1, idx_c, npad, d_hid, nch)

  g2 = _tc_layer2_in(p1, g1, dinv, b1.reshape(1, d_hid), W2, blk)

  p2 = _sc_segsum_1d(g2.reshape(npad), src_p, dst_p, npad)

  out = _tc_combine(p2.reshape(nc, npad, 1), g2, dinv,
                    b2.reshape(1, 1), blk)
  return out[:n]
